# Initial kernel scaffold; baseline (speedup 1.0000x reference)
import functools

import jax
import jax.numpy as jnp
from jax import lax
from jax.experimental import pallas as pl
from jax.experimental.pallas import tpu as pltpu
from jax.experimental.pallas import tpu_sc as plsc

N = 4096
D = 256
K = 32
NPAIR = 1024
ROWS = 256


def _knn_body(emb_blk_ref, emb_all_ref, out_ref):
    i = pl.program_id(0)
    eb = emb_blk_ref[...]
    ea = emb_all_ref[...]
    sq = lax.dot_general(
        jnp.ones((1, D), jnp.float32), ea * ea,
        (((1,), (1,)), ((), ())), preferred_element_type=jnp.float32)
    dot = lax.dot_general(
        eb, ea, (((1,), (1,)), ((), ())), preferred_element_type=jnp.float32)
    s = sq - 2.0 * dot
    cols = lax.broadcasted_iota(jnp.int32, (ROWS, N), 1)
    rows_g = lax.broadcasted_iota(jnp.int32, (ROWS, N), 0) + i * ROWS
    inf = jnp.float32(jnp.inf)
    s = jnp.where(cols == rows_g, inf, s)
    big = jnp.int32(2**30)
    idx_cols = []
    for _ in range(K):
        m = jnp.min(s, axis=1, keepdims=True)
        idx = jnp.min(jnp.where(s == m, cols, big), axis=1, keepdims=True)
        idx_cols.append(idx)
        s = jnp.where(cols == idx, inf, s)
    out_ref[...] = jnp.concatenate(idx_cols, axis=1)


_knn_call = pl.pallas_call(
    _knn_body,
    grid=(N // ROWS,),
    in_specs=[
        pl.BlockSpec((ROWS, D), lambda i: (i, 0)),
        pl.BlockSpec((N, D), lambda i: (0, 0)),
    ],
    out_specs=pl.BlockSpec((ROWS, K), lambda i: (i, 0)),
    out_shape=jax.ShapeDtypeStruct((N, K), jnp.int32),
)


NW = 32
PB = NPAIR // NW


@functools.partial(
    pl.kernel,
    out_type=[
        jax.ShapeDtypeStruct((NPAIR, K), jnp.int32),
        jax.ShapeDtypeStruct((NPAIR, K, K), jnp.int32),
    ],
    mesh=plsc.VectorSubcoreMesh(core_axis_name="c", subcore_axis_name="s"),
    scratch_types=[
        pltpu.VMEM((PB,), jnp.int32),
        pltpu.VMEM((PB, K), jnp.int32),
        pltpu.VMEM((PB, K, K), jnp.int32),
        pltpu.SemaphoreType.DMA,
        pltpu.SemaphoreType.DMA,
    ],
)
def _gather_call(knn_hbm, pair_hbm, a_out, g_out, pidx_v, a_v, g_v, sem, sem2):
    wid = lax.axis_index("s") * 2 + lax.axis_index("c")
    base = wid * PB
    pltpu.sync_copy(pair_hbm.at[pl.ds(base, PB)], pidx_v)
    pltpu.async_copy(knn_hbm.at[pidx_v], a_v, sem).wait()
    pltpu.sync_copy(a_v, a_out.at[pl.ds(base, PB)])
    copies = [
        pltpu.async_copy(knn_hbm.at[a_v.at[j]], g_v.at[j], sem2)
        for j in range(PB)
    ]
    for c in copies:
        c.wait()
    pltpu.sync_copy(g_v, g_out.at[pl.ds(base, PB)])


def _loss_body(a_ref, g_ref, r_ref, out_ref):
    a = a_ref[...]
    g = g_ref[...]
    acc = jnp.zeros((NPAIR, K * K), jnp.float32)
    for p in range(K):
        acc = acc + (g == a[:, p:p + 1]).astype(jnp.float32)
    c_i = lax.broadcasted_iota(jnp.int32, (K * K, K), 0) // K
    j_i = lax.broadcasted_iota(jnp.int32, (K * K, K), 1)
    sel = (c_i == j_i).astype(jnp.float32)
    counts = lax.dot_general(
        acc, sel, (((1,), (0,)), ((), ())), preferred_element_type=jnp.float32)
    cs = [jnp.sum((counts <= t).astype(jnp.float32), axis=1, keepdims=True)
          for t in range(K + 1)]
    cum = jnp.concatenate(cs, axis=1)
    ss = [jnp.sum((cum <= r).astype(jnp.float32), axis=1, keepdims=True)
          for r in range(K)]
    sorted_v = jnp.concatenate(ss, axis=1)
    dif = sorted_v * (1.0 / K) - r_ref[...]
    out_ref[0, 0] = jnp.sum(dif * dif) * (1.0 / (NPAIR * K))


_loss_call = pl.pallas_call(
    _loss_body,
    in_specs=[
        pl.BlockSpec((NPAIR, K), lambda: (0, 0)),
        pl.BlockSpec((NPAIR, K * K), lambda: (0, 0)),
        pl.BlockSpec((NPAIR, K), lambda: (0, 0)),
    ],
    out_specs=pl.BlockSpec(memory_space=pltpu.SMEM),
    out_shape=jax.ShapeDtypeStruct((1, 1), jnp.float32),
)


def kernel(embeddings, reference_second_order):
    knn = _knn_call(embeddings)
    pair = jax.random.permutation(jax.random.key(42), N)[:NPAIR].astype(jnp.int32)
    a, g = _gather_call(knn, pair)
    g2 = g.reshape(NPAIR, K * K)
    loss = _loss_call(a, g2, reference_second_order)
    return loss.reshape(())


# speedup vs baseline: 4.5401x; 4.5401x over previous
import functools

import jax
import jax.numpy as jnp
from jax import lax
from jax.experimental import pallas as pl
from jax.experimental.pallas import tpu as pltpu
from jax.experimental.pallas import tpu_sc as plsc

N = 4096
D = 256
K = 32
NPAIR = 1024
ROWS = 256


def _knn_body(emb_blk_ref, emb_all_ref, out_ref):
    i = pl.program_id(0)
    eb = emb_blk_ref[...]
    ea = emb_all_ref[...]
    sq = lax.dot_general(
        jnp.ones((1, D), jnp.float32), ea * ea,
        (((1,), (1,)), ((), ())), preferred_element_type=jnp.float32)
    dot = lax.dot_general(
        eb, ea, (((1,), (1,)), ((), ())), preferred_element_type=jnp.float32)
    s = sq - 2.0 * dot
    cols = lax.broadcasted_iota(jnp.int32, (ROWS, N), 1)
    rows_g = lax.broadcasted_iota(jnp.int32, (ROWS, N), 0) + i * ROWS
    inf = jnp.float32(jnp.inf)
    s = jnp.where(cols == rows_g, inf, s)
    big = jnp.int32(2**30)
    idx_cols = []
    for _ in range(K):
        m = jnp.min(s, axis=1, keepdims=True)
        idx = jnp.min(jnp.where(s == m, cols, big), axis=1, keepdims=True)
        idx_cols.append(idx)
        s = jnp.where(cols == idx, inf, s)
    pad = [jnp.full((ROWS, 128 - K), -1, jnp.int32)]
    out_ref[...] = jnp.concatenate(idx_cols + pad, axis=1)


_knn_call = pl.pallas_call(
    _knn_body,
    grid=(N // ROWS,),
    in_specs=[
        pl.BlockSpec((ROWS, D), lambda i: (i, 0)),
        pl.BlockSpec((N, D), lambda i: (0, 0)),
    ],
    out_specs=pl.BlockSpec((ROWS, 128), lambda i: (i, 0)),
    out_shape=jax.ShapeDtypeStruct((N, 128), jnp.int32),
)


NW = 32
PB = NPAIR // NW


@functools.cache
def _make_gather_call():
    @functools.partial(
        pl.kernel,
        out_type=[
            jax.ShapeDtypeStruct((NPAIR, 128), jnp.int32),
            jax.ShapeDtypeStruct((NPAIR, K, 128), jnp.int32),
        ],
        mesh=plsc.VectorSubcoreMesh(core_axis_name="c", subcore_axis_name="s"),
        scratch_types=[
            pltpu.VMEM((PB,), jnp.int32),
            pltpu.VMEM((PB, 128), jnp.int32),
            pltpu.VMEM((8, K, 128), jnp.int32),
            pltpu.SemaphoreType.DMA,
            pltpu.SemaphoreType.DMA,
        ],
    )
    def _gather(knn_hbm, pair_hbm, a_out, g_out, pidx_v, a_v, g_v, sem, sem2):
        wid = lax.axis_index("s") * 2 + lax.axis_index("c")
        base = wid * PB
        pltpu.sync_copy(pair_hbm.at[pl.ds(base, PB)], pidx_v)
        pltpu.async_copy(knn_hbm.at[pidx_v], a_v, sem).wait()
        pltpu.sync_copy(a_v, a_out.at[pl.ds(base, PB)])
        for c in range(PB // 8):
            copies = [
                pltpu.async_copy(
                    knn_hbm.at[a_v.at[c * 8 + j, pl.ds(0, K)]],
                    g_v.at[j], sem2)
                for j in range(8)
            ]
            for cp in copies:
                cp.wait()
            pltpu.sync_copy(g_v, g_out.at[pl.ds(base + c * 8, 8)])

    return _gather


LB = 256


def _loss_body(a_ref, g_ref, r_ref, out_ref):
    i = pl.program_id(0)
    a = a_ref[...]
    g = g_ref[...]
    counts = jnp.zeros((LB, K), jnp.float32)
    for p in range(K):
        a_p = jnp.reshape(a[:, p], (LB, 1, 1))
        counts = counts + jnp.sum((g == a_p).astype(jnp.float32), axis=2)
    cs = [jnp.sum((counts <= t).astype(jnp.float32), axis=1, keepdims=True)
          for t in range(K + 1)]
    cum = jnp.concatenate(cs, axis=1)
    ss = [jnp.sum((cum <= r).astype(jnp.float32), axis=1, keepdims=True)
          for r in range(K)]
    sorted_v = jnp.concatenate(ss, axis=1)
    dif = sorted_v * (1.0 / K) - r_ref[...]
    part = jnp.sum(dif * dif) * (1.0 / (NPAIR * K))

    @pl.when(i == 0)
    def _():
        out_ref[0, 0] = 0.0

    out_ref[0, 0] += part


_loss_call = pl.pallas_call(
    _loss_body,
    grid=(NPAIR // LB,),
    in_specs=[
        pl.BlockSpec((LB, 128), lambda i: (i, 0)),
        pl.BlockSpec((LB, K, 128), lambda i: (i, 0, 0)),
        pl.BlockSpec((LB, K), lambda i: (i, 0)),
    ],
    out_specs=pl.BlockSpec(memory_space=pltpu.SMEM),
    out_shape=jax.ShapeDtypeStruct((1, 1), jnp.float32),
)


def kernel(embeddings, reference_second_order):
    knn_pad = _knn_call(embeddings, embeddings)
    pair = jax.random.permutation(jax.random.key(42), N)[:NPAIR].astype(jnp.int32)
    a, g = _make_gather_call()(knn_pad, pair)
    loss = _loss_call(a, g, reference_second_order)
    return loss.reshape(())


# speedup vs baseline: 5.8216x; 1.2822x over previous
import functools

import jax
import jax.numpy as jnp
from jax import lax
from jax.experimental import pallas as pl
from jax.experimental.pallas import tpu as pltpu
from jax.experimental.pallas import tpu_sc as plsc

N = 4096
D = 256
K = 32
NPAIR = 1024
ROWS = 256


def _knn_body(emb_blk_ref, emb_all_ref, out_ref):
    i = pl.program_id(0)
    eb = emb_blk_ref[...]
    ea = emb_all_ref[...]
    sq = lax.dot_general(
        jnp.ones((1, D), jnp.float32), ea * ea,
        (((1,), (1,)), ((), ())), preferred_element_type=jnp.float32)
    dot = lax.dot_general(
        eb, ea, (((1,), (1,)), ((), ())), preferred_element_type=jnp.float32)
    s = sq - 2.0 * dot
    cols = lax.broadcasted_iota(jnp.int32, (ROWS, N), 1)
    rows_g = lax.broadcasted_iota(jnp.int32, (ROWS, N), 0) + i * ROWS
    b = lax.bitcast_convert_type(s, jnp.int32)
    k = b ^ (lax.shift_right_arithmetic(b, 31) & jnp.int32(0x7FFFFFFF))
    k = (k & jnp.int32(-4096)) | cols
    imax = jnp.int32(0x7FFFFFFF)
    k = jnp.where(cols == rows_g, imax, k)
    m = jnp.min(k, axis=1, keepdims=True)
    idx_cols = [m & 4095]
    for _ in range(K - 1):
        m = jnp.min(jnp.where(k > m, k, imax), axis=1, keepdims=True)
        idx_cols.append(m & 4095)
    pad = [jnp.full((ROWS, 128 - K), -1, jnp.int32)]
    out_ref[...] = jnp.concatenate(idx_cols + pad, axis=1)


_knn_call = pl.pallas_call(
    _knn_body,
    grid=(N // ROWS,),
    in_specs=[
        pl.BlockSpec((ROWS, D), lambda i: (i, 0)),
        pl.BlockSpec((N, D), lambda i: (0, 0)),
    ],
    out_specs=pl.BlockSpec((ROWS, 128), lambda i: (i, 0)),
    out_shape=jax.ShapeDtypeStruct((N, 128), jnp.int32),
)


NW = 32
PB = NPAIR // NW


@functools.cache
def _make_gather_call():
    @functools.partial(
        pl.kernel,
        out_type=[
            jax.ShapeDtypeStruct((NPAIR, 128), jnp.int32),
            jax.ShapeDtypeStruct((NPAIR, K, 128), jnp.int32),
        ],
        mesh=plsc.VectorSubcoreMesh(core_axis_name="c", subcore_axis_name="s"),
        scratch_types=[
            pltpu.VMEM((PB,), jnp.int32),
            pltpu.VMEM((PB, 128), jnp.int32),
            pltpu.VMEM((8, K, 128), jnp.int32),
            pltpu.SemaphoreType.DMA,
            pltpu.SemaphoreType.DMA,
        ],
    )
    def _gather(knn_hbm, pair_hbm, a_out, g_out, pidx_v, a_v, g_v, sem, sem2):
        wid = lax.axis_index("s") * 2 + lax.axis_index("c")
        base = wid * PB
        pltpu.sync_copy(pair_hbm.at[pl.ds(base, PB)], pidx_v)
        pltpu.async_copy(knn_hbm.at[pidx_v], a_v, sem).wait()
        pltpu.sync_copy(a_v, a_out.at[pl.ds(base, PB)])
        for c in range(PB // 8):
            copies = [
                pltpu.async_copy(
                    knn_hbm.at[a_v.at[c * 8 + j, pl.ds(0, K)]],
                    g_v.at[j], sem2)
                for j in range(8)
            ]
            for cp in copies:
                cp.wait()
            pltpu.sync_copy(g_v, g_out.at[pl.ds(base + c * 8, 8)])

    return _gather


LB = 256


def _loss_body(a_ref, g_ref, r_ref, out_ref):
    i = pl.program_id(0)
    a = a_ref[...]
    g = g_ref[...]
    counts = jnp.zeros((LB, K), jnp.float32)
    for p in range(K):
        a_p = jnp.reshape(a[:, p], (LB, 1, 1))
        counts = counts + jnp.sum((g == a_p).astype(jnp.float32), axis=2)
    cs = [jnp.sum((counts <= t).astype(jnp.float32), axis=1, keepdims=True)
          for t in range(K + 1)]
    cum = jnp.concatenate(cs, axis=1)
    ss = [jnp.sum((cum <= r).astype(jnp.float32), axis=1, keepdims=True)
          for r in range(K)]
    sorted_v = jnp.concatenate(ss, axis=1)
    dif = sorted_v * (1.0 / K) - r_ref[...]
    part = jnp.sum(dif * dif) * (1.0 / (NPAIR * K))

    @pl.when(i == 0)
    def _():
        out_ref[0, 0] = 0.0

    out_ref[0, 0] += part


_loss_call = pl.pallas_call(
    _loss_body,
    grid=(NPAIR // LB,),
    in_specs=[
        pl.BlockSpec((LB, 128), lambda i: (i, 0)),
        pl.BlockSpec((LB, K, 128), lambda i: (i, 0, 0)),
        pl.BlockSpec((LB, K), lambda i: (i, 0)),
    ],
    out_specs=pl.BlockSpec(memory_space=pltpu.SMEM),
    out_shape=jax.ShapeDtypeStruct((1, 1), jnp.float32),
)


def kernel(embeddings, reference_second_order):
    knn_pad = _knn_call(embeddings, embeddings)
    pair = jax.random.permutation(jax.random.key(42), N)[:NPAIR].astype(jnp.int32)
    a, g = _make_gather_call()(knn_pad, pair)
    loss = _loss_call(a, g, reference_second_order)
    return loss.reshape(())


# speedup vs baseline: 9.9796x; 1.7142x over previous
import functools

import numpy as _np

import jax
import jax.numpy as jnp
from jax import lax
from jax.experimental import pallas as pl
from jax.experimental.pallas import tpu as pltpu
from jax.experimental.pallas import tpu_sc as plsc

N = 4096
D = 256
K = 32
NPAIR = 1024
ROWS = 256


def _knn_body(emb_blk_ref, emb_all_ref, out_ref):
    i = pl.program_id(0)
    eb = emb_blk_ref[...]
    ea = emb_all_ref[...]
    sq = lax.dot_general(
        jnp.ones((1, D), jnp.float32), ea * ea,
        (((1,), (1,)), ((), ())), preferred_element_type=jnp.float32)
    dot = lax.dot_general(
        eb, ea, (((1,), (1,)), ((), ())), preferred_element_type=jnp.float32)
    s = sq - 2.0 * dot
    cols = lax.broadcasted_iota(jnp.int32, (ROWS, N), 1)
    rows_g = lax.broadcasted_iota(jnp.int32, (ROWS, N), 0) + i * ROWS
    b = lax.bitcast_convert_type(s, jnp.int32)
    k = b ^ (lax.shift_right_arithmetic(b, 31) & jnp.int32(0x7FFFFFFF))
    k = (k & jnp.int32(-4096)) | cols
    imax = jnp.int32(0x7FFFFFFF)
    k = jnp.where(cols == rows_g, imax, k)
    m = jnp.min(k, axis=1, keepdims=True)
    idx_cols = [m & 4095]
    for _ in range(K - 1):
        m = jnp.min(jnp.where(k > m, k, imax), axis=1, keepdims=True)
        idx_cols.append(m & 4095)
    out_ref[...] = jnp.concatenate(idx_cols, axis=1)


_knn_call = pl.pallas_call(
    _knn_body,
    grid=(N // ROWS,),
    in_specs=[
        pl.BlockSpec((ROWS, D), lambda i: (i, 0)),
        pl.BlockSpec((N, D), lambda i: (0, 0)),
    ],
    out_specs=pl.BlockSpec((ROWS, K), lambda i: (i, 0)),
    out_shape=jax.ShapeDtypeStruct((N, K), jnp.int32),
)


NW = 32
PB = NPAIR // NW


@functools.cache
def _make_gather_call():
    @functools.partial(
        pl.kernel,
        out_type=[
            jax.ShapeDtypeStruct((NPAIR, K), jnp.int32),
            jax.ShapeDtypeStruct((NPAIR, K, K), jnp.int32),
        ],
        mesh=plsc.VectorSubcoreMesh(core_axis_name="c", subcore_axis_name="s"),
        compiler_params=pltpu.CompilerParams(use_tc_tiling_on_sc=False),
        scratch_types=[
            pltpu.VMEM((PB,), jnp.int32),
            pltpu.VMEM((PB, K), jnp.int32),
            pltpu.VMEM((PB, K, K), jnp.int32),
            pltpu.SemaphoreType.DMA,
            pltpu.SemaphoreType.DMA,
        ],
    )
    def _gather(knn_hbm, pair_hbm, a_out, g_out, pidx_v, a_v, g_v, sem, sem2):
        wid = lax.axis_index("s") * 2 + lax.axis_index("c")
        base = wid * PB
        pltpu.sync_copy(pair_hbm.at[pl.ds(base, PB)], pidx_v)
        pltpu.async_copy(knn_hbm.at[pidx_v], a_v, sem).wait()
        pltpu.sync_copy(a_v, a_out.at[pl.ds(base, PB)])
        for c in range(PB // 8):
            copies = [
                pltpu.async_copy(
                    knn_hbm.at[a_v.at[c * 8 + j]],
                    g_v.at[c * 8 + j], sem2)
                for j in range(8)
            ]
            for cp in copies:
                cp.wait()
        pltpu.sync_copy(g_v, g_out.at[pl.ds(base, PB)])

    return _gather


def _loss_body(a_ref, g_ref, r_ref, out_ref):
    a = a_ref[...]
    g = g_ref[...]
    acc = jnp.zeros((NPAIR, K * K), jnp.float32)
    for p in range(K):
        acc = acc + (g == a[:, p:p + 1]).astype(jnp.float32)
    c_i = lax.broadcasted_iota(jnp.int32, (K * K, K), 0) // K
    j_i = lax.broadcasted_iota(jnp.int32, (K * K, K), 1)
    sel = (c_i == j_i).astype(jnp.float32)
    counts = lax.dot_general(
        acc, sel, (((1,), (0,)), ((), ())), preferred_element_type=jnp.float32)
    cs = [jnp.sum((counts <= t).astype(jnp.float32), axis=1, keepdims=True)
          for t in range(K + 1)]
    cum = jnp.concatenate(cs, axis=1)
    ss = [jnp.sum((cum <= r).astype(jnp.float32), axis=1, keepdims=True)
          for r in range(K)]
    sorted_v = jnp.concatenate(ss, axis=1)
    dif = sorted_v * (1.0 / K) - r_ref[...]
    out_ref[0, 0] = jnp.sum(dif * dif) * (1.0 / (NPAIR * K))


_loss_call = pl.pallas_call(
    _loss_body,
    in_specs=[
        pl.BlockSpec((NPAIR, K), lambda: (0, 0)),
        pl.BlockSpec((NPAIR, K * K), lambda: (0, 0)),
        pl.BlockSpec((NPAIR, K), lambda: (0, 0)),
    ],
    out_specs=pl.BlockSpec(memory_space=pltpu.SMEM),
    out_shape=jax.ShapeDtypeStruct((1, 1), jnp.float32),
)


_PAIR_IDX = _np.asarray(
    jax.random.permutation(jax.random.key(42), N)[:NPAIR]).astype(_np.int32)


def kernel(embeddings, reference_second_order):
    knn = _knn_call(embeddings, embeddings)
    a, g = _make_gather_call()(knn, jnp.asarray(_PAIR_IDX))
    g2 = g.reshape(NPAIR, K * K)
    loss = _loss_call(a, g2, reference_second_order)
    return loss.reshape(())
